# baseline (speedup 1.0000x reference)
import jax
import jax.numpy as jnp
from jax import lax
from jax.experimental import pallas as pl
from jax.experimental.pallas import tpu as pltpu
from jax.experimental.pallas import tpu_sc as plsc

N = 10000
NPAD = 10240
E = 320000
F = 128
NC, NS = 2, 16
NW = NC * NS
EPT = E // NW
CHUNK = 80
NCHUNK = EPT // CHUNK
_GOFF = (0, 24, 48, 64)
_GLEN = (24, 24, 16, 16)
GSPLIT = len(_GOFF)
ROWS_PT = NPAD // NS

_f32 = jnp.float32


_DK = 4
_DG = NCHUNK // _DK
_DTAIL = NCHUNK - _DG * _DK


def _deg_body(dst2d_hbm, out_hbm, ones_v, dall_v, isem, csems, acc_sh):
    cid = lax.axis_index("c")
    sid = lax.axis_index("s")
    wid = sid * NC + cid
    base = sid * ROWS_PT

    pltpu.async_copy(dst2d_hbm.at[wid], dall_v, isem)

    def _zl(i, c):
        ones_v[pl.ds(i * 16, 16)] = jnp.zeros((16,), _f32)
        return c
    lax.fori_loop(0, CHUNK // 16, _zl, 0)
    for j in range(ROWS_PT // CHUNK):
        pltpu.sync_copy(ones_v, acc_sh.at[pl.ds(base + j * CHUNK, CHUNK)])

    def _ol(i, c):
        ones_v[pl.ds(i * 16, 16)] = jnp.ones((16,), _f32)
        return c
    lax.fori_loop(0, CHUNK // 16, _ol, 0)
    pltpu.make_async_copy(dst2d_hbm.at[wid], dall_v, isem).wait()
    plsc.subcore_barrier()

    def fire_scatter(i, b):
        pltpu.async_copy(ones_v, acc_sh.at[dall_v.at[i]], csems[b], add=True)

    def wait_scatter(i, b):
        pltpu.make_async_copy(ones_v, acc_sh.at[dall_v.at[i]],
                              csems[b]).wait()

    def _group(g, c):
        for b in range(_DK):
            @pl.when(g > 0)
            def _(b=b):
                wait_scatter(0, b)
            fire_scatter(g * _DK + b, b)
        return c
    lax.fori_loop(0, _DG, _group, 0)
    for b in range(_DK):
        wait_scatter(0, b)
    for t in range(_DTAIL):
        pltpu.sync_copy(ones_v, acc_sh.at[dall_v.at[_DG * _DK + t]],
                        add=True)
    plsc.subcore_barrier()
    pltpu.sync_copy(acc_sh.at[pl.ds(base, ROWS_PT)],
                    out_hbm.at[cid, pl.ds(base, ROWS_PT)])


import functools


@functools.lru_cache(maxsize=None)
def _sc_calls():
    mesh = plsc.VectorSubcoreMesh(
        core_axis_name="c", subcore_axis_name="s",
        num_cores=NC, num_subcores=NS)
    deg_call = pl.kernel(
        _deg_body,
        out_type=jax.ShapeDtypeStruct((NC, NPAD), _f32),
        mesh=mesh,
        scratch_types=[
            pltpu.VMEM((CHUNK,), _f32),
            pltpu.VMEM((NCHUNK, CHUNK), jnp.int32),
            pltpu.SemaphoreType.DMA,
            [pltpu.SemaphoreType.DMA] * _DK,
            pltpu.VMEM_SHARED((NPAD,), _f32),
        ],
    )
    agg_call = pl.kernel(
        _agg_body,
        out_type=jax.ShapeDtypeStruct((NC, N, F), _f32),
        mesh=mesh,
        scratch_types=[
            pltpu.VMEM((EPT,), jnp.int32),
            pltpu.VMEM((NCHUNK, CHUNK), jnp.int32),
            [pltpu.VMEM((CHUNK, F), _f32)] * 2,
            [pltpu.SemaphoreType.DMA] * 2,
            [pltpu.SemaphoreType.DMA] * (2 * GSPLIT),
            [pltpu.SemaphoreType.DMA] * 2,
            pltpu.VMEM_SHARED((N, F), _f32),
        ],
    )
    return deg_call, agg_call


def _agg_body(src1d_hbm, dst3d_hbm, table_hbm, out_hbm,
              sall_v, dall_v, rbufs, isems, gsems, csems, acc_sh):
    cid = lax.axis_index("c")
    sid = lax.axis_index("s")
    wid = sid * NC + cid
    base = sid * ROWS_PT

    pltpu.async_copy(src1d_hbm.at[pl.ds(wid * EPT, EPT)], sall_v, isems[0])
    pltpu.async_copy(dst3d_hbm.at[wid], dall_v, isems[1])

    def fire_gather(i, b):
        for k in range(GSPLIT):
            pltpu.async_copy(
                table_hbm.at[sall_v.at[pl.ds(i * CHUNK + _GOFF[k], _GLEN[k])]],
                rbufs[b].at[pl.ds(_GOFF[k], _GLEN[k])],
                gsems[b * GSPLIT + k])

    def wait_gather(i, b):
        for k in range(GSPLIT):
            pltpu.make_async_copy(
                table_hbm.at[sall_v.at[pl.ds(i * CHUNK + _GOFF[k], _GLEN[k])]],
                rbufs[b].at[pl.ds(_GOFF[k], _GLEN[k])],
                gsems[b * GSPLIT + k]).wait()

    def fire_scatter(i, b):
        pltpu.async_copy(rbufs[b], acc_sh.at[dall_v.at[i]], csems[b],
                         add=True)

    def wait_scatter(i, b):
        pltpu.make_async_copy(rbufs[b], acc_sh.at[dall_v.at[i]],
                              csems[b]).wait()

    pltpu.make_async_copy(src1d_hbm.at[pl.ds(wid * EPT, EPT)], sall_v,
                          isems[0]).wait()
    fire_gather(0, 0)
    @pl.when(sid < NS - 1)
    def _():
        pltpu.sync_copy(table_hbm.at[pl.ds(base, ROWS_PT)],
                        acc_sh.at[pl.ds(base, ROWS_PT), :])

    @pl.when(sid == NS - 1)
    def _():
        pltpu.sync_copy(table_hbm.at[pl.ds((NS - 1) * ROWS_PT,
                                           N - (NS - 1) * ROWS_PT)],
                        acc_sh.at[pl.ds((NS - 1) * ROWS_PT,
                                        N - (NS - 1) * ROWS_PT), :])
    pltpu.make_async_copy(dst3d_hbm.at[wid], dall_v, isems[1]).wait()
    plsc.subcore_barrier()

    def _pair(j, c):
        i0 = 2 * j
        wait_gather(i0, 0)

        @pl.when(j > 0)
        def _():
            wait_scatter(i0 - 1, 1)
        fire_gather(i0 + 1, 1)
        fire_scatter(i0, 0)
        wait_gather(i0 + 1, 1)
        wait_scatter(i0, 0)
        fire_gather(i0 + 2, 0)
        fire_scatter(i0 + 1, 1)
        return c
    lax.fori_loop(0, (NCHUNK - 1) // 2, _pair, 0)
    wait_gather(NCHUNK - 1, 0)
    wait_scatter(NCHUNK - 2, 1)
    fire_scatter(NCHUNK - 1, 0)
    wait_scatter(NCHUNK - 1, 0)
    plsc.subcore_barrier()

    @pl.when(sid < NS - 1)
    def _():
        pltpu.sync_copy(acc_sh.at[pl.ds(base, ROWS_PT), :],
                        out_hbm.at[cid, pl.ds(base, ROWS_PT), :])

    @pl.when(sid == NS - 1)
    def _():
        pltpu.sync_copy(
            acc_sh.at[pl.ds((NS - 1) * ROWS_PT, N - (NS - 1) * ROWS_PT), :],
            out_hbm.at[cid, pl.ds((NS - 1) * ROWS_PT,
                                  N - (NS - 1) * ROWS_PT), :])


_BLK = 1000
_GRID = N // _BLK


def _mm_scale_body(d0_ref, d1_ref, x_ref, w_ref, h1p_ref, dis_ref):
    deg = 1.0 + d0_ref[0] + d1_ref[0]
    dis = lax.rsqrt(deg)
    h = jnp.dot(x_ref[...], w_ref[...], preferred_element_type=_f32)
    h1p_ref[...] = h * dis
    dis_ref[...] = dis


_mm_scale = pl.pallas_call(
    _mm_scale_body,
    grid=(_GRID,),
    in_specs=[
        pl.BlockSpec((1, _BLK, 1), lambda i: (0, i, 0)),
        pl.BlockSpec((1, _BLK, 1), lambda i: (1, i, 0)),
        pl.BlockSpec((_BLK, F), lambda i: (i, 0)),
        pl.BlockSpec((F, F), lambda i: (0, 0)),
    ],
    out_specs=[
        pl.BlockSpec((_BLK, F), lambda i: (i, 0)),
        pl.BlockSpec((_BLK, 1), lambda i: (i, 0)),
    ],
    out_shape=[
        jax.ShapeDtypeStruct((N, F), _f32),
        jax.ShapeDtypeStruct((N, 1), _f32),
    ],
)


def _mid_body(p0_ref, p1_ref, h1p_ref, x_ref, dis_ref, w_ref, b_ref, out_ref):
    dis = dis_ref[...]
    conv = dis * (p0_ref[0] + p1_ref[0] - h1p_ref[...]) + b_ref[...]
    h = jnp.maximum(conv, 0.0) + x_ref[...]
    out_ref[...] = dis * jnp.dot(h, w_ref[...], preferred_element_type=_f32)


_mid = pl.pallas_call(
    _mid_body,
    grid=(_GRID,),
    in_specs=[
        pl.BlockSpec((1, _BLK, F), lambda i: (0, i, 0)),
        pl.BlockSpec((1, _BLK, F), lambda i: (1, i, 0)),
        pl.BlockSpec((_BLK, F), lambda i: (i, 0)),
        pl.BlockSpec((_BLK, F), lambda i: (i, 0)),
        pl.BlockSpec((_BLK, 1), lambda i: (i, 0)),
        pl.BlockSpec((F, F), lambda i: (0, 0)),
        pl.BlockSpec((1, F), lambda i: (0, 0)),
    ],
    out_specs=pl.BlockSpec((_BLK, F), lambda i: (i, 0)),
    out_shape=jax.ShapeDtypeStruct((N, F), _f32),
)


def _final_body(q0_ref, q1_ref, h2p_ref, dis_ref, b_ref, out_ref):
    out_ref[...] = (dis_ref[...] * (q0_ref[0] + q1_ref[0] - h2p_ref[...])
                    + b_ref[...])


_final = pl.pallas_call(
    _final_body,
    grid=(_GRID,),
    in_specs=[
        pl.BlockSpec((1, _BLK, F), lambda i: (0, i, 0)),
        pl.BlockSpec((1, _BLK, F), lambda i: (1, i, 0)),
        pl.BlockSpec((_BLK, F), lambda i: (i, 0)),
        pl.BlockSpec((_BLK, 1), lambda i: (i, 0)),
        pl.BlockSpec((1, F), lambda i: (0, 0)),
    ],
    out_specs=pl.BlockSpec((_BLK, F), lambda i: (i, 0)),
    out_shape=jax.ShapeDtypeStruct((N, F), _f32),
)


def kernel(edge_index, x, W1, b1, W2, b2):
    _deg_call, _agg_call = _sc_calls()
    src1d = edge_index[0]
    dst3d = edge_index[1].reshape(NW, NCHUNK, CHUNK)
    degp = _deg_call(dst3d)
    degp3 = degp.reshape(NC, NPAD, 1)
    h1p, dis = _mm_scale(degp3, degp3, x, W1)
    p = _agg_call(src1d, dst3d, h1p)
    h2p = _mid(p, p, h1p, x, dis, W2, b1.reshape(1, F))
    q = _agg_call(src1d, dst3d, h2p)
    return _final(q, q, h2p, dis, b2.reshape(1, F))


# speedup vs baseline: 1.0071x; 1.0071x over previous
import jax
import jax.numpy as jnp
from jax import lax
from jax.experimental import pallas as pl
from jax.experimental.pallas import tpu as pltpu
from jax.experimental.pallas import tpu_sc as plsc

N = 10000
NPAD = 10240
E = 320000
F = 128
NC, NS = 2, 16
NW = NC * NS
EPT = E // NW
CHUNK = 80
NCHUNK = EPT // CHUNK
_GOFF = (0, 24, 48, 64)
_GLEN = (24, 24, 16, 16)
GSPLIT = len(_GOFF)
ROWS_PT = NPAD // NS

_f32 = jnp.float32


_DK = 4
_DG = NCHUNK // _DK
_DTAIL = NCHUNK - _DG * _DK


def _deg_body(dst2d_hbm, out_hbm, ones_v, dall_v, isem, csems, acc_sh):
    cid = lax.axis_index("c")
    sid = lax.axis_index("s")
    wid = sid * NC + cid
    base = sid * ROWS_PT

    pltpu.async_copy(dst2d_hbm.at[wid], dall_v, isem)

    def _zl(i, c):
        ones_v[pl.ds(i * 16, 16)] = jnp.zeros((16,), _f32)
        return c
    lax.fori_loop(0, CHUNK // 16, _zl, 0)
    for j in range(ROWS_PT // CHUNK):
        pltpu.sync_copy(ones_v, acc_sh.at[pl.ds(base + j * CHUNK, CHUNK)])

    def _ol(i, c):
        ones_v[pl.ds(i * 16, 16)] = jnp.ones((16,), _f32)
        return c
    lax.fori_loop(0, CHUNK // 16, _ol, 0)
    pltpu.make_async_copy(dst2d_hbm.at[wid], dall_v, isem).wait()
    plsc.subcore_barrier()

    def fire_scatter(i, b):
        pltpu.async_copy(ones_v, acc_sh.at[dall_v.at[i]], csems[b], add=True)

    def wait_scatter(i, b):
        pltpu.make_async_copy(ones_v, acc_sh.at[dall_v.at[i]],
                              csems[b]).wait()

    def _group(g, c):
        for b in range(_DK):
            @pl.when(g > 0)
            def _(b=b):
                wait_scatter(0, b)
            fire_scatter(g * _DK + b, b)
        return c
    lax.fori_loop(0, _DG, _group, 0)
    for b in range(_DK):
        wait_scatter(0, b)
    for t in range(_DTAIL):
        pltpu.sync_copy(ones_v, acc_sh.at[dall_v.at[_DG * _DK + t]],
                        add=True)
    plsc.subcore_barrier()
    pltpu.sync_copy(acc_sh.at[pl.ds(base, ROWS_PT)],
                    out_hbm.at[cid, pl.ds(base, ROWS_PT)])


import functools


@functools.lru_cache(maxsize=None)
def _sc_calls():
    mesh = plsc.VectorSubcoreMesh(
        core_axis_name="c", subcore_axis_name="s",
        num_cores=NC, num_subcores=NS)
    deg_call = pl.kernel(
        _deg_body,
        out_type=jax.ShapeDtypeStruct((NC, NPAD), _f32),
        mesh=mesh,
        scratch_types=[
            pltpu.VMEM((CHUNK,), _f32),
            pltpu.VMEM((NCHUNK, CHUNK), jnp.int32),
            pltpu.SemaphoreType.DMA,
            [pltpu.SemaphoreType.DMA] * _DK,
            pltpu.VMEM_SHARED((NPAD,), _f32),
        ],
    )
    agg_call = pl.kernel(
        _agg_body,
        out_type=jax.ShapeDtypeStruct((NC, N, F), _f32),
        mesh=mesh,
        compiler_params=pltpu.CompilerParams(use_tc_tiling_on_sc=True),
        scratch_types=[
            pltpu.VMEM((EPT,), jnp.int32),
            pltpu.VMEM((NCHUNK, CHUNK), jnp.int32),
            [pltpu.VMEM((CHUNK, F), _f32)] * 2,
            [pltpu.SemaphoreType.DMA] * 2,
            [pltpu.SemaphoreType.DMA] * (2 * GSPLIT),
            [pltpu.SemaphoreType.DMA] * 2,
            pltpu.VMEM_SHARED((N, F), _f32),
        ],
    )
    return deg_call, agg_call


def _agg_body(src1d_hbm, dst3d_hbm, table_hbm, out_hbm,
              sall_v, dall_v, rbufs, isems, gsems, csems, acc_sh):
    cid = lax.axis_index("c")
    sid = lax.axis_index("s")
    wid = sid * NC + cid
    base = sid * ROWS_PT

    pltpu.async_copy(src1d_hbm.at[pl.ds(wid * EPT, EPT)], sall_v, isems[0])
    pltpu.async_copy(dst3d_hbm.at[wid], dall_v, isems[1])

    def fire_gather(i, b):
        for k in range(GSPLIT):
            pltpu.async_copy(
                table_hbm.at[sall_v.at[pl.ds(i * CHUNK + _GOFF[k], _GLEN[k])]],
                rbufs[b].at[pl.ds(_GOFF[k], _GLEN[k])],
                gsems[b * GSPLIT + k])

    def wait_gather(i, b):
        for k in range(GSPLIT):
            pltpu.make_async_copy(
                table_hbm.at[sall_v.at[pl.ds(i * CHUNK + _GOFF[k], _GLEN[k])]],
                rbufs[b].at[pl.ds(_GOFF[k], _GLEN[k])],
                gsems[b * GSPLIT + k]).wait()

    def fire_scatter(i, b):
        pltpu.async_copy(rbufs[b], acc_sh.at[dall_v.at[i]], csems[b],
                         add=True)

    def wait_scatter(i, b):
        pltpu.make_async_copy(rbufs[b], acc_sh.at[dall_v.at[i]],
                              csems[b]).wait()

    @pl.when(sid < NS - 1)
    def _():
        pltpu.sync_copy(table_hbm.at[pl.ds(base, ROWS_PT)],
                        acc_sh.at[pl.ds(base, ROWS_PT), :])

    @pl.when(sid == NS - 1)
    def _():
        pltpu.sync_copy(table_hbm.at[pl.ds((NS - 1) * ROWS_PT,
                                           N - (NS - 1) * ROWS_PT)],
                        acc_sh.at[pl.ds((NS - 1) * ROWS_PT,
                                        N - (NS - 1) * ROWS_PT), :])
    pltpu.make_async_copy(src1d_hbm.at[pl.ds(wid * EPT, EPT)], sall_v,
                          isems[0]).wait()
    pltpu.make_async_copy(dst3d_hbm.at[wid], dall_v, isems[1]).wait()
    fire_gather(0, 0)
    plsc.subcore_barrier()

    def _pair(j, c):
        i0 = 2 * j
        wait_gather(i0, 0)

        @pl.when(j > 0)
        def _():
            wait_scatter(i0 - 1, 1)
        fire_gather(i0 + 1, 1)
        fire_scatter(i0, 0)
        wait_gather(i0 + 1, 1)
        wait_scatter(i0, 0)
        fire_gather(i0 + 2, 0)
        fire_scatter(i0 + 1, 1)
        return c
    lax.fori_loop(0, (NCHUNK - 1) // 2, _pair, 0)
    wait_gather(NCHUNK - 1, 0)
    wait_scatter(NCHUNK - 2, 1)
    fire_scatter(NCHUNK - 1, 0)
    wait_scatter(NCHUNK - 1, 0)
    plsc.subcore_barrier()

    @pl.when(sid < NS - 1)
    def _():
        pltpu.sync_copy(acc_sh.at[pl.ds(base, ROWS_PT), :],
                        out_hbm.at[cid, pl.ds(base, ROWS_PT), :])

    @pl.when(sid == NS - 1)
    def _():
        pltpu.sync_copy(
            acc_sh.at[pl.ds((NS - 1) * ROWS_PT, N - (NS - 1) * ROWS_PT), :],
            out_hbm.at[cid, pl.ds((NS - 1) * ROWS_PT,
                                  N - (NS - 1) * ROWS_PT), :])


_BLK = 1000
_GRID = N // _BLK


def _mm_scale_body(d0_ref, d1_ref, x_ref, w_ref, h1p_ref, dis_ref):
    deg = 1.0 + d0_ref[0] + d1_ref[0]
    dis = lax.rsqrt(deg)
    h = jnp.dot(x_ref[...], w_ref[...], preferred_element_type=_f32)
    h1p_ref[...] = h * dis
    dis_ref[...] = dis


_mm_scale = pl.pallas_call(
    _mm_scale_body,
    grid=(_GRID,),
    in_specs=[
        pl.BlockSpec((1, _BLK, 1), lambda i: (0, i, 0)),
        pl.BlockSpec((1, _BLK, 1), lambda i: (1, i, 0)),
        pl.BlockSpec((_BLK, F), lambda i: (i, 0)),
        pl.BlockSpec((F, F), lambda i: (0, 0)),
    ],
    out_specs=[
        pl.BlockSpec((_BLK, F), lambda i: (i, 0)),
        pl.BlockSpec((_BLK, 1), lambda i: (i, 0)),
    ],
    out_shape=[
        jax.ShapeDtypeStruct((N, F), _f32),
        jax.ShapeDtypeStruct((N, 1), _f32),
    ],
)


def _mid_body(p0_ref, p1_ref, h1p_ref, x_ref, dis_ref, w_ref, b_ref, out_ref):
    dis = dis_ref[...]
    conv = dis * (p0_ref[0] + p1_ref[0] - h1p_ref[...]) + b_ref[...]
    h = jnp.maximum(conv, 0.0) + x_ref[...]
    out_ref[...] = dis * jnp.dot(h, w_ref[...], preferred_element_type=_f32)


_mid = pl.pallas_call(
    _mid_body,
    grid=(_GRID,),
    in_specs=[
        pl.BlockSpec((1, _BLK, F), lambda i: (0, i, 0)),
        pl.BlockSpec((1, _BLK, F), lambda i: (1, i, 0)),
        pl.BlockSpec((_BLK, F), lambda i: (i, 0)),
        pl.BlockSpec((_BLK, F), lambda i: (i, 0)),
        pl.BlockSpec((_BLK, 1), lambda i: (i, 0)),
        pl.BlockSpec((F, F), lambda i: (0, 0)),
        pl.BlockSpec((1, F), lambda i: (0, 0)),
    ],
    out_specs=pl.BlockSpec((_BLK, F), lambda i: (i, 0)),
    out_shape=jax.ShapeDtypeStruct((N, F), _f32),
)


def _final_body(q0_ref, q1_ref, h2p_ref, dis_ref, b_ref, out_ref):
    out_ref[...] = (dis_ref[...] * (q0_ref[0] + q1_ref[0] - h2p_ref[...])
                    + b_ref[...])


_final = pl.pallas_call(
    _final_body,
    grid=(_GRID,),
    in_specs=[
        pl.BlockSpec((1, _BLK, F), lambda i: (0, i, 0)),
        pl.BlockSpec((1, _BLK, F), lambda i: (1, i, 0)),
        pl.BlockSpec((_BLK, F), lambda i: (i, 0)),
        pl.BlockSpec((_BLK, 1), lambda i: (i, 0)),
        pl.BlockSpec((1, F), lambda i: (0, 0)),
    ],
    out_specs=pl.BlockSpec((_BLK, F), lambda i: (i, 0)),
    out_shape=jax.ShapeDtypeStruct((N, F), _f32),
)


def kernel(edge_index, x, W1, b1, W2, b2):
    _deg_call, _agg_call = _sc_calls()
    src1d = edge_index[0]
    dst3d = edge_index[1].reshape(NW, NCHUNK, CHUNK)
    degp = _deg_call(dst3d)
    degp3 = degp.reshape(NC, NPAD, 1)
    h1p, dis = _mm_scale(degp3, degp3, x, W1)
    p = _agg_call(src1d, dst3d, h1p)
    h2p = _mid(p, p, h1p, x, dis, W2, b1.reshape(1, F))
    q = _agg_call(src1d, dst3d, h2p)
    return _final(q, q, h2p, dis, b2.reshape(1, F))
